# Initial kernel scaffold; baseline (speedup 1.0000x reference)
import jax
import jax.numpy as jnp
from jax.experimental import pallas as pl


def kernel(x_src, x_dst, edge_index, edge_attr, enc_W1, enc_b1, enc_W2, enc_b2, enc_W3, enc_b3, enc_g, enc_bn, node_W1, node_b1, node_W2, node_b2, node_W3, node_b3, node_g, node_bn, edge_W1, edge_b1, edge_W2, edge_b2, edge_W3, edge_b3, edge_g, edge_bn):
    raise NotImplementedError("write your pallas kernel here")


# speedup vs baseline: 2.9266x; 2.9266x over previous
import functools

import jax
import jax.numpy as jnp
from jax import lax
from jax.experimental import pallas as pl
from jax.experimental.pallas import tpu as pltpu
from jax.experimental.pallas import tpu_sc as plsc

N = 10000
E = 320000
D = 128
ED = 16

NC = 2
NS = 16
NW = NC * NS
CHUNK = 80
NCH_TILE = (E // NW) // CHUNK
E_TILE = CHUNK * NCH_TILE
ROWS_TILE = 632
NPAD = NS * ROWS_TILE

BE = 2000
BN = 2000


def _full_spec(shape):
    return pl.BlockSpec(shape, lambda i: tuple(0 for _ in shape))


def _silu(x):
    return x * jax.nn.sigmoid(x)


def _ln(h, g, bn):
    mu = jnp.mean(h, axis=-1, keepdims=True)
    var = jnp.var(h, axis=-1, keepdims=True)
    return (h - mu) * lax.rsqrt(var + 1e-5) * g + bn


def _project(x_dst, x_src, W1a, W1b, b1):
    nb = N // BN

    def body(xd_ref, xs_ref, wa_ref, wb_ref, b1_ref, out_ref):
        out_ref[0] = (
            jnp.dot(xd_ref[...], wa_ref[...], preferred_element_type=jnp.float32)
            + b1_ref[...]
        )
        out_ref[1] = jnp.dot(
            xs_ref[...], wb_ref[...], preferred_element_type=jnp.float32
        )

    return pl.pallas_call(
        body,
        grid=(nb,),
        in_specs=[
            pl.BlockSpec((BN, D), lambda i: (i, 0)),
            pl.BlockSpec((BN, D), lambda i: (i, 0)),
            _full_spec((D, D)),
            _full_spec((D, D)),
            _full_spec((1, D)),
        ],
        out_specs=pl.BlockSpec((2, BN, D), lambda i: (0, i, 0)),
        out_shape=jax.ShapeDtypeStruct((2, N, D), jnp.float32),
    )(x_dst, x_src, W1a, W1b, b1)


def _gather_sum(table, dix, six):
    mesh = plsc.VectorSubcoreMesh(core_axis_name="c", subcore_axis_name="s")

    @functools.partial(
        pl.kernel,
        out_type=jax.ShapeDtypeStruct((E, D), jnp.float32),
        mesh=mesh,
        scratch_types=[
            pltpu.VMEM((NCH_TILE, CHUNK), jnp.int32),
            pltpu.VMEM((NCH_TILE, CHUNK), jnp.int32),
            pltpu.VMEM((CHUNK, D), jnp.float32),
            pltpu.SemaphoreType.DMA,
        ],
    )
    def k(table_hbm, dix_hbm, six_hbm, out_hbm, dv, sv, buf, sem):
        cid = lax.axis_index("c")
        sid = lax.axis_index("s")
        wid = sid * NC + cid
        pltpu.sync_copy(dix_hbm.at[wid], dv)
        pltpu.sync_copy(six_hbm.at[wid], sv)

        def body(j, carry):
            pltpu.async_copy(table_hbm.at[dv.at[j]], buf, sem).wait()
            pltpu.async_copy(table_hbm.at[sv.at[j]], buf, sem, add=True).wait()
            pltpu.async_copy(
                buf, out_hbm.at[pl.ds(wid * E_TILE + j * CHUNK, CHUNK)], sem
            ).wait()
            return carry

        lax.fori_loop(0, NCH_TILE, body, 0)

    return k(table, dix, six)


def _scatter_add(e_new, dix, zeros_pad):
    mesh = plsc.VectorSubcoreMesh(core_axis_name="c", subcore_axis_name="s")

    @functools.partial(
        pl.kernel,
        out_type=jax.ShapeDtypeStruct((NC, NPAD, D), jnp.float32),
        mesh=mesh,
        scratch_types=[
            pltpu.VMEM((NCH_TILE, CHUNK), jnp.int32),
            pltpu.VMEM((CHUNK, D), jnp.float32),
            pltpu.VMEM_SHARED((NPAD, D), jnp.float32),
            pltpu.SemaphoreType.DMA,
        ],
    )
    def k(e_hbm, dix_hbm, z_hbm, out_hbm, dv, buf, acc, sem):
        cid = lax.axis_index("c")
        sid = lax.axis_index("s")
        wid = sid * NC + cid
        zr0 = sid * ROWS_TILE
        pltpu.sync_copy(z_hbm.at[pl.ds(zr0, ROWS_TILE)], acc.at[pl.ds(zr0, ROWS_TILE)])
        pltpu.sync_copy(dix_hbm.at[wid], dv)
        plsc.subcore_barrier()

        def body(j, carry):
            pltpu.async_copy(
                e_hbm.at[pl.ds(wid * E_TILE + j * CHUNK, CHUNK)], buf, sem
            ).wait()
            pltpu.sync_copy(buf, acc.at[dv.at[j]], add=True)
            return carry

        lax.fori_loop(0, NCH_TILE, body, 0)
        plsc.subcore_barrier()
        pltpu.sync_copy(
            acc.at[pl.ds(zr0, ROWS_TILE)], out_hbm.at[cid, pl.ds(zr0, ROWS_TILE)]
        )

    return k(e_new, dix, zeros_pad)


def _edge_mlp(s, ea_or_attr, enc_ws, W1c, W2, W3, b2, b3, g, bn, first_layer):
    nb = E // BE

    def body(*refs):
        if first_layer:
            (s_ref, a_ref, eW1, eb1, eW2, eb2, eW3, eb3, eg, ebn,
             w1c, w2, w3, b2r, b3r, gr, bnr, out_ref) = refs
            t = jnp.dot(a_ref[...], eW1[...], preferred_element_type=jnp.float32)
            t = _silu(t + eb1[...])
            t = _silu(jnp.dot(t, eW2[...], preferred_element_type=jnp.float32)
                      + eb2[...])
            t = jnp.dot(t, eW3[...], preferred_element_type=jnp.float32) + eb3[...]
            ea = _ln(t, eg[...], ebn[...])
        else:
            (s_ref, a_ref, w1c, w2, w3, b2r, b3r, gr, bnr, out_ref) = refs
            ea = a_ref[...]
        a = s_ref[...] + jnp.dot(ea, w1c[...], preferred_element_type=jnp.float32)
        h = _silu(a)
        h = _silu(jnp.dot(h, w2[...], preferred_element_type=jnp.float32) + b2r[...])
        h = jnp.dot(h, w3[...], preferred_element_type=jnp.float32) + b3r[...]
        out_ref[...] = _ln(h, gr[...], bnr[...]) + ea

    ea_dim = ED if first_layer else D
    in_specs = [
        pl.BlockSpec((BE, D), lambda i: (i, 0)),
        pl.BlockSpec((BE, ea_dim), lambda i: (i, 0)),
    ]
    args = [s, ea_or_attr]
    if first_layer:
        in_specs += [
            _full_spec((ED, D)), _full_spec((1, D)),
            _full_spec((D, D)), _full_spec((1, D)),
            _full_spec((D, D)), _full_spec((1, D)),
            _full_spec((1, D)), _full_spec((1, D)),
        ]
        args += list(enc_ws)
    in_specs += [
        _full_spec((D, D)), _full_spec((D, D)), _full_spec((D, D)),
        _full_spec((1, D)), _full_spec((1, D)),
        _full_spec((1, D)), _full_spec((1, D)),
    ]
    args += [W1c, W2, W3, b2, b3, g, bn]

    return pl.pallas_call(
        body,
        grid=(nb,),
        in_specs=in_specs,
        out_specs=pl.BlockSpec((BE, D), lambda i: (i, 0)),
        out_shape=jax.ShapeDtypeStruct((E, D), jnp.float32),
    )(*args)


def _node_mlp(x, parts, W1, b1, W2, b2, W3, b3, g, bn):
    nb = N // BN

    def body(x_ref, p_ref, w1_ref, b1r, w2, b2r, w3, b3r, gr, bnr, out_ref):
        xb = x_ref[...]
        agg = p_ref[0] + p_ref[1]
        a = (jnp.dot(xb, w1_ref[0:D, :], preferred_element_type=jnp.float32)
             + jnp.dot(agg, w1_ref[D:2 * D, :], preferred_element_type=jnp.float32)
             + b1r[...])
        h = _silu(a)
        h = _silu(jnp.dot(h, w2[...], preferred_element_type=jnp.float32) + b2r[...])
        h = jnp.dot(h, w3[...], preferred_element_type=jnp.float32) + b3r[...]
        out_ref[...] = _ln(h, gr[...], bnr[...]) + xb

    return pl.pallas_call(
        body,
        grid=(nb,),
        in_specs=[
            pl.BlockSpec((BN, D), lambda i: (i, 0)),
            pl.BlockSpec((NC, BN, D), lambda i: (0, i, 0)),
            _full_spec((2 * D, D)),
            _full_spec((1, D)),
            _full_spec((D, D)),
            _full_spec((1, D)),
            _full_spec((D, D)),
            _full_spec((1, D)),
            _full_spec((1, D)),
            _full_spec((1, D)),
        ],
        out_specs=pl.BlockSpec((BN, D), lambda i: (i, 0)),
        out_shape=jax.ShapeDtypeStruct((N, D), jnp.float32),
    )(x, parts, W1, b1, W2, b2, W3, b3, g, bn)


def kernel(x_src, x_dst, edge_index, edge_attr,
           enc_W1, enc_b1, enc_W2, enc_b2, enc_W3, enc_b3, enc_g, enc_bn,
           node_W1, node_b1, node_W2, node_b2, node_W3, node_b3, node_g, node_bn,
           edge_W1, edge_b1, edge_W2, edge_b2, edge_W3, edge_b3, edge_g, edge_bn):
    row = lambda v: v.reshape(1, D)
    src = edge_index[0]
    dst = edge_index[1]
    dix = dst.reshape(NW, NCH_TILE, CHUNK)
    six = (src + N).reshape(NW, NCH_TILE, CHUNK)
    zeros_pad = jnp.zeros((NPAD, D), jnp.float32)
    enc_ws = (enc_W1, row(enc_b1), enc_W2, row(enc_b2), enc_W3, row(enc_b3),
              row(enc_g), row(enc_bn))

    ea = edge_attr
    for i in range(2):
        W1 = edge_W1[i]
        proj = _project(x_dst, x_src, W1[0:D, :], W1[D:2 * D, :],
                        row(edge_b1[i]))
        s = _gather_sum(proj.reshape(2 * N, D), dix, six)
        e_new = _edge_mlp(
            s, ea, enc_ws, W1[2 * D:3 * D, :], edge_W2[i], edge_W3[i],
            row(edge_b2[i]), row(edge_b3[i]), row(edge_g[i]), row(edge_bn[i]),
            first_layer=(i == 0),
        )
        parts = _scatter_add(e_new, dix, zeros_pad)
        x_dst = _node_mlp(
            x_dst, parts[:, 0:N, :], node_W1[i], row(node_b1[i]),
            node_W2[i], row(node_b2[i]), node_W3[i], row(node_b3[i]),
            row(node_g[i]), row(node_bn[i]),
        )
        ea = e_new
    return x_dst


# speedup vs baseline: 4.0746x; 1.3923x over previous
import functools

import jax
import jax.numpy as jnp
from jax import lax
from jax.experimental import pallas as pl
from jax.experimental.pallas import tpu as pltpu
from jax.experimental.pallas import tpu_sc as plsc

N = 10000
E = 320000
D = 128
ED = 16

NC = 2
NS = 16
NW = NC * NS
CHUNK = 80
NCH_TILE = (E // NW) // CHUNK
E_TILE = CHUNK * NCH_TILE
ROWS_TILE = 632
NPAD = NS * ROWS_TILE

BE = 2000
BN = 2000


def _full_spec(shape):
    return pl.BlockSpec(shape, lambda i: tuple(0 for _ in shape))


def _silu(x):
    return x * jax.nn.sigmoid(x)


def _ln(h, g, bn):
    mu = jnp.mean(h, axis=-1, keepdims=True)
    var = jnp.var(h, axis=-1, keepdims=True)
    return (h - mu) * lax.rsqrt(var + 1e-5) * g + bn


def _project(x_dst, x_src, W1a, W1b, b1):
    nb = N // BN

    def body(xd_ref, xs_ref, wa_ref, wb_ref, b1_ref, out_ref):
        out_ref[0] = (
            jnp.dot(xd_ref[...], wa_ref[...], preferred_element_type=jnp.float32)
            + b1_ref[...]
        )
        out_ref[1] = jnp.dot(
            xs_ref[...], wb_ref[...], preferred_element_type=jnp.float32
        )

    return pl.pallas_call(
        body,
        grid=(nb,),
        in_specs=[
            pl.BlockSpec((BN, D), lambda i: (i, 0)),
            pl.BlockSpec((BN, D), lambda i: (i, 0)),
            _full_spec((D, D)),
            _full_spec((D, D)),
            _full_spec((1, D)),
        ],
        out_specs=pl.BlockSpec((2, BN, D), lambda i: (0, i, 0)),
        out_shape=jax.ShapeDtypeStruct((2, N, D), jnp.float32),
    )(x_dst, x_src, W1a, W1b, b1)


KB = 5
CH_G = 40
NCH_G = E_TILE // CH_G
NIT_G = NCH_G // KB


def _gather_sum(table, dix, six):
    mesh = plsc.VectorSubcoreMesh(core_axis_name="c", subcore_axis_name="s")

    @functools.partial(
        pl.kernel,
        out_type=jax.ShapeDtypeStruct((E, D), jnp.float32),
        mesh=mesh,
        scratch_types=[
            pltpu.VMEM((NCH_G, CH_G), jnp.int32),
            pltpu.VMEM((NCH_G, CH_G), jnp.int32),
            pltpu.VMEM((2 * KB, CH_G, D), jnp.float32),
            pltpu.SemaphoreType.DMA,
            pltpu.SemaphoreType.DMA,
        ],
    )
    def k(table_hbm, dix_hbm, six_hbm, out_hbm, dv, sv, bufs, sem, semw):
        cid = lax.axis_index("c")
        sid = lax.axis_index("s")
        wid = sid * NC + cid
        pltpu.sync_copy(dix_hbm.at[wid], dv)
        pltpu.sync_copy(six_hbm.at[wid], sv)

        def body(j, carry):
            bank = (j % 2) * KB

            @pl.when(j >= 2)
            def _():
                for b in range(KB):
                    pltpu.make_async_copy(
                        bufs.at[bank + b], out_hbm.at[pl.ds(0, CH_G)], semw
                    ).wait()

            gd = [
                pltpu.async_copy(
                    table_hbm.at[dv.at[j * KB + b]], bufs.at[bank + b], sem
                )
                for b in range(KB)
            ]
            ga = []
            for b in range(KB):
                gd[b].wait()
                ga.append(
                    pltpu.async_copy(
                        table_hbm.at[sv.at[j * KB + b]], bufs.at[bank + b],
                        sem, add=True,
                    )
                )
            for b in range(KB):
                ga[b].wait()
                pltpu.async_copy(
                    bufs.at[bank + b],
                    out_hbm.at[pl.ds(wid * E_TILE + (j * KB + b) * CH_G, CH_G)],
                    semw,
                )
            return carry

        lax.fori_loop(0, NIT_G, body, 0)
        for _ in range(2 * KB):
            pltpu.make_async_copy(
                bufs.at[0], out_hbm.at[pl.ds(0, CH_G)], semw
            ).wait()

    return k(table, dix, six)


def _scatter_add(e_new, dix, zeros_pad):
    mesh = plsc.VectorSubcoreMesh(core_axis_name="c", subcore_axis_name="s")

    @functools.partial(
        pl.kernel,
        out_type=jax.ShapeDtypeStruct((NC, NPAD, D), jnp.float32),
        mesh=mesh,
        scratch_types=[
            pltpu.VMEM((NCH_TILE, CHUNK), jnp.int32),
            pltpu.VMEM((3, CHUNK, D), jnp.float32),
            pltpu.VMEM_SHARED((NPAD, D), jnp.float32),
            pltpu.SemaphoreType.DMA,
            pltpu.SemaphoreType.DMA,
        ],
    )
    def k(e_hbm, dix_hbm, z_hbm, out_hbm, dv, bufs, acc, semr, sems):
        cid = lax.axis_index("c")
        sid = lax.axis_index("s")
        wid = sid * NC + cid
        zr0 = sid * ROWS_TILE
        pltpu.sync_copy(z_hbm.at[pl.ds(zr0, ROWS_TILE)], acc.at[pl.ds(zr0, ROWS_TILE)])
        pltpu.sync_copy(dix_hbm.at[wid], dv)
        plsc.subcore_barrier()

        e0 = wid * E_TILE
        pltpu.async_copy(e_hbm.at[pl.ds(e0, CHUNK)], bufs.at[0], semr)

        def body(j, carry):
            @pl.when(j >= 2)
            def _():
                pltpu.make_async_copy(bufs.at[0], acc.at[dv.at[0]], sems).wait()

            @pl.when(j + 1 < NCH_TILE)
            def _():
                pltpu.async_copy(
                    e_hbm.at[pl.ds(e0 + (j + 1) * CHUNK, CHUNK)],
                    bufs.at[(j + 1) % 3], semr,
                )

            pltpu.make_async_copy(e_hbm.at[pl.ds(0, CHUNK)], bufs.at[0], semr).wait()
            pltpu.async_copy(bufs.at[j % 3], acc.at[dv.at[j]], sems, add=True)
            return carry

        lax.fori_loop(0, NCH_TILE, body, 0)
        for _ in range(2):
            pltpu.make_async_copy(bufs.at[0], acc.at[dv.at[0]], sems).wait()
        plsc.subcore_barrier()
        pltpu.sync_copy(
            acc.at[pl.ds(zr0, ROWS_TILE)], out_hbm.at[cid, pl.ds(zr0, ROWS_TILE)]
        )

    return k(e_new, dix, zeros_pad)


def _edge_mlp(s, ea_or_attr, enc_ws, W1c, W2, W3, b2, b3, g, bn, first_layer):
    nb = E // BE

    def body(*refs):
        if first_layer:
            (s_ref, a_ref, eW1, eb1, eW2, eb2, eW3, eb3, eg, ebn,
             w1c, w2, w3, b2r, b3r, gr, bnr, out_ref) = refs
            t = jnp.dot(a_ref[...], eW1[...], preferred_element_type=jnp.float32)
            t = _silu(t + eb1[...])
            t = _silu(jnp.dot(t, eW2[...], preferred_element_type=jnp.float32)
                      + eb2[...])
            t = jnp.dot(t, eW3[...], preferred_element_type=jnp.float32) + eb3[...]
            ea = _ln(t, eg[...], ebn[...])
        else:
            (s_ref, a_ref, w1c, w2, w3, b2r, b3r, gr, bnr, out_ref) = refs
            ea = a_ref[...]
        a = s_ref[...] + jnp.dot(ea, w1c[...], preferred_element_type=jnp.float32)
        h = _silu(a)
        h = _silu(jnp.dot(h, w2[...], preferred_element_type=jnp.float32) + b2r[...])
        h = jnp.dot(h, w3[...], preferred_element_type=jnp.float32) + b3r[...]
        out_ref[...] = _ln(h, gr[...], bnr[...]) + ea

    ea_dim = ED if first_layer else D
    in_specs = [
        pl.BlockSpec((BE, D), lambda i: (i, 0)),
        pl.BlockSpec((BE, ea_dim), lambda i: (i, 0)),
    ]
    args = [s, ea_or_attr]
    if first_layer:
        in_specs += [
            _full_spec((ED, D)), _full_spec((1, D)),
            _full_spec((D, D)), _full_spec((1, D)),
            _full_spec((D, D)), _full_spec((1, D)),
            _full_spec((1, D)), _full_spec((1, D)),
        ]
        args += list(enc_ws)
    in_specs += [
        _full_spec((D, D)), _full_spec((D, D)), _full_spec((D, D)),
        _full_spec((1, D)), _full_spec((1, D)),
        _full_spec((1, D)), _full_spec((1, D)),
    ]
    args += [W1c, W2, W3, b2, b3, g, bn]

    return pl.pallas_call(
        body,
        grid=(nb,),
        in_specs=in_specs,
        out_specs=pl.BlockSpec((BE, D), lambda i: (i, 0)),
        out_shape=jax.ShapeDtypeStruct((E, D), jnp.float32),
    )(*args)


def _node_mlp(x, parts, W1, b1, W2, b2, W3, b3, g, bn):
    nb = N // BN

    def body(x_ref, p_ref, w1_ref, b1r, w2, b2r, w3, b3r, gr, bnr, out_ref):
        xb = x_ref[...]
        agg = p_ref[0] + p_ref[1]
        a = (jnp.dot(xb, w1_ref[0:D, :], preferred_element_type=jnp.float32)
             + jnp.dot(agg, w1_ref[D:2 * D, :], preferred_element_type=jnp.float32)
             + b1r[...])
        h = _silu(a)
        h = _silu(jnp.dot(h, w2[...], preferred_element_type=jnp.float32) + b2r[...])
        h = jnp.dot(h, w3[...], preferred_element_type=jnp.float32) + b3r[...]
        out_ref[...] = _ln(h, gr[...], bnr[...]) + xb

    return pl.pallas_call(
        body,
        grid=(nb,),
        in_specs=[
            pl.BlockSpec((BN, D), lambda i: (i, 0)),
            pl.BlockSpec((NC, BN, D), lambda i: (0, i, 0)),
            _full_spec((2 * D, D)),
            _full_spec((1, D)),
            _full_spec((D, D)),
            _full_spec((1, D)),
            _full_spec((D, D)),
            _full_spec((1, D)),
            _full_spec((1, D)),
            _full_spec((1, D)),
        ],
        out_specs=pl.BlockSpec((BN, D), lambda i: (i, 0)),
        out_shape=jax.ShapeDtypeStruct((N, D), jnp.float32),
    )(x, parts, W1, b1, W2, b2, W3, b3, g, bn)


def kernel(x_src, x_dst, edge_index, edge_attr,
           enc_W1, enc_b1, enc_W2, enc_b2, enc_W3, enc_b3, enc_g, enc_bn,
           node_W1, node_b1, node_W2, node_b2, node_W3, node_b3, node_g, node_bn,
           edge_W1, edge_b1, edge_W2, edge_b2, edge_W3, edge_b3, edge_g, edge_bn):
    row = lambda v: v.reshape(1, D)
    src = edge_index[0]
    dst = edge_index[1]
    dix_g = dst.reshape(NW, NCH_G, CH_G)
    six_g = (src + N).reshape(NW, NCH_G, CH_G)
    dix = dst.reshape(NW, NCH_TILE, CHUNK)
    zeros_pad = jnp.zeros((NPAD, D), jnp.float32)
    enc_ws = (enc_W1, row(enc_b1), enc_W2, row(enc_b2), enc_W3, row(enc_b3),
              row(enc_g), row(enc_bn))

    ea = edge_attr
    for i in range(2):
        W1 = edge_W1[i]
        proj = _project(x_dst, x_src, W1[0:D, :], W1[D:2 * D, :],
                        row(edge_b1[i]))
        s = _gather_sum(proj.reshape(2 * N, D), dix_g, six_g)
        e_new = _edge_mlp(
            s, ea, enc_ws, W1[2 * D:3 * D, :], edge_W2[i], edge_W3[i],
            row(edge_b2[i]), row(edge_b3[i]), row(edge_g[i]), row(edge_bn[i]),
            first_layer=(i == 0),
        )
        parts = _scatter_add(e_new, dix, zeros_pad)
        x_dst = _node_mlp(
            x_dst, parts[:, 0:N, :], node_W1[i], row(node_b1[i]),
            node_W2[i], row(node_b2[i]), node_W3[i], row(node_b3[i]),
            row(node_g[i]), row(node_bn[i]),
        )
        ea = e_new
    return x_dst


# speedup vs baseline: 4.3755x; 1.0739x over previous
import functools

import jax
import jax.numpy as jnp
from jax import lax
from jax.experimental import pallas as pl
from jax.experimental.pallas import tpu as pltpu
from jax.experimental.pallas import tpu_sc as plsc

N = 10000
E = 320000
D = 128
ED = 16

NC = 2
NS = 16
NW = NC * NS

NSP = 2
ESP = E // NSP
ET = ESP // NW
CH = 40
NCH = ET // CH
KB = 5
NIT = NCH // KB

ROWS_TILE = 632
NPAD = NS * ROWS_TILE

BE = 2000
BN = 2000


def _full_spec(shape):
    return pl.BlockSpec(shape, lambda i: tuple(0 for _ in shape))


def _silu(x):
    return x * jax.nn.sigmoid(x)


def _ln(h, g, bn):
    mu = jnp.mean(h, axis=-1, keepdims=True)
    var = jnp.var(h, axis=-1, keepdims=True)
    return (h - mu) * lax.rsqrt(var + 1e-5) * g + bn


def _project(x_dst, x_src, W1a, W1b, b1):
    nb = N // BN

    def body(xd_ref, xs_ref, wa_ref, wb_ref, b1_ref, out_ref):
        out_ref[0] = (
            jnp.dot(xd_ref[...], wa_ref[...], preferred_element_type=jnp.float32)
            + b1_ref[...]
        )
        out_ref[1] = jnp.dot(
            xs_ref[...], wb_ref[...], preferred_element_type=jnp.float32
        )

    return pl.pallas_call(
        body,
        grid=(nb,),
        in_specs=[
            pl.BlockSpec((BN, D), lambda i: (i, 0)),
            pl.BlockSpec((BN, D), lambda i: (i, 0)),
            _full_spec((D, D)),
            _full_spec((D, D)),
            _full_spec((1, D)),
        ],
        out_specs=pl.BlockSpec((2, BN, D), lambda i: (0, i, 0)),
        out_shape=jax.ShapeDtypeStruct((2, N, D), jnp.float32),
    )(x_dst, x_src, W1a, W1b, b1)


def _gather_sum(table, dix, six):
    mesh = plsc.VectorSubcoreMesh(core_axis_name="c", subcore_axis_name="s")

    @functools.partial(
        pl.kernel,
        out_type=jax.ShapeDtypeStruct((ESP, D), jnp.float32),
        mesh=mesh,
        scratch_types=[
            pltpu.VMEM((NCH, CH), jnp.int32),
            pltpu.VMEM((NCH, CH), jnp.int32),
            pltpu.VMEM((2 * KB, CH, D), jnp.float32),
            pltpu.SemaphoreType.DMA,
            pltpu.SemaphoreType.DMA,
        ],
    )
    def k(table_hbm, dix_hbm, six_hbm, out_hbm, dv, sv, bufs, sem, semw):
        cid = lax.axis_index("c")
        sid = lax.axis_index("s")
        wid = sid * NC + cid
        pltpu.sync_copy(dix_hbm.at[wid], dv)
        pltpu.sync_copy(six_hbm.at[wid], sv)

        def body(j, carry):
            bank = (j % 2) * KB

            @pl.when(j >= 2)
            def _():
                for b in range(KB):
                    pltpu.make_async_copy(
                        bufs.at[bank + b], out_hbm.at[pl.ds(0, CH)], semw
                    ).wait()

            gd = [
                pltpu.async_copy(
                    table_hbm.at[dv.at[j * KB + b]], bufs.at[bank + b], sem
                )
                for b in range(KB)
            ]
            ga = []
            for b in range(KB):
                gd[b].wait()
                ga.append(
                    pltpu.async_copy(
                        table_hbm.at[sv.at[j * KB + b]], bufs.at[bank + b],
                        sem, add=True,
                    )
                )
            for b in range(KB):
                ga[b].wait()
                pltpu.async_copy(
                    bufs.at[bank + b],
                    out_hbm.at[pl.ds(wid * ET + (j * KB + b) * CH, CH)],
                    semw,
                )
            return carry

        lax.fori_loop(0, NIT, body, 0)
        for _ in range(2 * KB):
            pltpu.make_async_copy(
                bufs.at[0], out_hbm.at[pl.ds(0, CH)], semw
            ).wait()

    return k(table, dix, six)


def _scatter_add(e_new, dix, zeros_pad):
    mesh = plsc.VectorSubcoreMesh(core_axis_name="c", subcore_axis_name="s")

    @functools.partial(
        pl.kernel,
        out_type=jax.ShapeDtypeStruct((NC, NPAD, D), jnp.float32),
        mesh=mesh,
        scratch_types=[
            pltpu.VMEM((NCH, CH), jnp.int32),
            pltpu.VMEM((3, CH, D), jnp.float32),
            pltpu.VMEM_SHARED((NPAD, D), jnp.float32),
            pltpu.SemaphoreType.DMA,
            pltpu.SemaphoreType.DMA,
        ],
    )
    def k(e_hbm, dix_hbm, z_hbm, out_hbm, dv, bufs, acc, semr, sems):
        cid = lax.axis_index("c")
        sid = lax.axis_index("s")
        wid = sid * NC + cid
        zr0 = sid * ROWS_TILE
        pltpu.sync_copy(z_hbm.at[pl.ds(zr0, ROWS_TILE)], acc.at[pl.ds(zr0, ROWS_TILE)])
        pltpu.sync_copy(dix_hbm.at[wid], dv)
        plsc.subcore_barrier()

        e0 = wid * ET
        pltpu.async_copy(e_hbm.at[pl.ds(e0, CH)], bufs.at[0], semr)

        def body(j, carry):
            @pl.when(j >= 2)
            def _():
                pltpu.make_async_copy(bufs.at[0], acc.at[dv.at[0]], sems).wait()

            @pl.when(j + 1 < NCH)
            def _():
                pltpu.async_copy(
                    e_hbm.at[pl.ds(e0 + (j + 1) * CH, CH)],
                    bufs.at[(j + 1) % 3], semr,
                )

            pltpu.make_async_copy(e_hbm.at[pl.ds(0, CH)], bufs.at[0], semr).wait()
            pltpu.async_copy(bufs.at[j % 3], acc.at[dv.at[j]], sems, add=True)
            return carry

        lax.fori_loop(0, NCH, body, 0)
        for _ in range(2):
            pltpu.make_async_copy(bufs.at[0], acc.at[dv.at[0]], sems).wait()
        plsc.subcore_barrier()
        pltpu.sync_copy(
            acc.at[pl.ds(zr0, ROWS_TILE)], out_hbm.at[cid, pl.ds(zr0, ROWS_TILE)]
        )

    return k(e_new, dix, zeros_pad)


def _edge_mlp(s, ea_or_attr, enc_ws, W1c, W2, W3, b2, b3, g, bn, first_layer):
    nb = ESP // BE

    def body(*refs):
        if first_layer:
            (s_ref, a_ref, eW1, eb1, eW2, eb2, eW3, eb3, eg, ebn,
             w1c, w2, w3, b2r, b3r, gr, bnr, out_ref) = refs
            t = jnp.dot(a_ref[...], eW1[...], preferred_element_type=jnp.float32)
            t = _silu(t + eb1[...])
            t = _silu(jnp.dot(t, eW2[...], preferred_element_type=jnp.float32)
                      + eb2[...])
            t = jnp.dot(t, eW3[...], preferred_element_type=jnp.float32) + eb3[...]
            ea = _ln(t, eg[...], ebn[...])
        else:
            (s_ref, a_ref, w1c, w2, w3, b2r, b3r, gr, bnr, out_ref) = refs
            ea = a_ref[...]
        a = s_ref[...] + jnp.dot(ea, w1c[...], preferred_element_type=jnp.float32)
        h = _silu(a)
        h = _silu(jnp.dot(h, w2[...], preferred_element_type=jnp.float32) + b2r[...])
        h = jnp.dot(h, w3[...], preferred_element_type=jnp.float32) + b3r[...]
        out_ref[...] = _ln(h, gr[...], bnr[...]) + ea

    ea_dim = ED if first_layer else D
    in_specs = [
        pl.BlockSpec((BE, D), lambda i: (i, 0)),
        pl.BlockSpec((BE, ea_dim), lambda i: (i, 0)),
    ]
    args = [s, ea_or_attr]
    if first_layer:
        in_specs += [
            _full_spec((ED, D)), _full_spec((1, D)),
            _full_spec((D, D)), _full_spec((1, D)),
            _full_spec((D, D)), _full_spec((1, D)),
            _full_spec((1, D)), _full_spec((1, D)),
        ]
        args += list(enc_ws)
    in_specs += [
        _full_spec((D, D)), _full_spec((D, D)), _full_spec((D, D)),
        _full_spec((1, D)), _full_spec((1, D)),
        _full_spec((1, D)), _full_spec((1, D)),
    ]
    args += [W1c, W2, W3, b2, b3, g, bn]

    return pl.pallas_call(
        body,
        grid=(nb,),
        in_specs=in_specs,
        out_specs=pl.BlockSpec((BE, D), lambda i: (i, 0)),
        out_shape=jax.ShapeDtypeStruct((ESP, D), jnp.float32),
    )(*args)


def _node_mlp(x, parts_list, W1, b1, W2, b2, W3, b3, g, bn):
    nb = N // BN

    def body(x_ref, p0_ref, p1_ref, w1_ref, b1r, w2, b2r, w3, b3r, gr, bnr,
             out_ref):
        xb = x_ref[...]
        agg = (p0_ref[0] + p0_ref[1]) + (p1_ref[0] + p1_ref[1])
        a = (jnp.dot(xb, w1_ref[0:D, :], preferred_element_type=jnp.float32)
             + jnp.dot(agg, w1_ref[D:2 * D, :], preferred_element_type=jnp.float32)
             + b1r[...])
        h = _silu(a)
        h = _silu(jnp.dot(h, w2[...], preferred_element_type=jnp.float32) + b2r[...])
        h = jnp.dot(h, w3[...], preferred_element_type=jnp.float32) + b3r[...]
        out_ref[...] = _ln(h, gr[...], bnr[...]) + xb

    part_spec = pl.BlockSpec((NC, BN, D), lambda i: (0, i, 0))
    return pl.pallas_call(
        body,
        grid=(nb,),
        in_specs=[
            pl.BlockSpec((BN, D), lambda i: (i, 0)),
            part_spec, part_spec,
            _full_spec((2 * D, D)),
            _full_spec((1, D)),
            _full_spec((D, D)),
            _full_spec((1, D)),
            _full_spec((D, D)),
            _full_spec((1, D)),
            _full_spec((1, D)),
            _full_spec((1, D)),
        ],
        out_specs=pl.BlockSpec((BN, D), lambda i: (i, 0)),
        out_shape=jax.ShapeDtypeStruct((N, D), jnp.float32),
    )(x, parts_list[0], parts_list[1], W1, b1, W2, b2, W3, b3, g, bn)


def kernel(x_src, x_dst, edge_index, edge_attr,
           enc_W1, enc_b1, enc_W2, enc_b2, enc_W3, enc_b3, enc_g, enc_bn,
           node_W1, node_b1, node_W2, node_b2, node_W3, node_b3, node_g, node_bn,
           edge_W1, edge_b1, edge_W2, edge_b2, edge_W3, edge_b3, edge_g, edge_bn):
    row = lambda v: v.reshape(1, D)
    src = edge_index[0]
    dst = edge_index[1]
    dix = [dst[h * ESP:(h + 1) * ESP].reshape(NW, NCH, CH) for h in range(NSP)]
    six = [(src[h * ESP:(h + 1) * ESP] + N).reshape(NW, NCH, CH)
           for h in range(NSP)]
    attr = [edge_attr[h * ESP:(h + 1) * ESP] for h in range(NSP)]
    zeros_pad = jnp.zeros((NPAD, D), jnp.float32)
    enc_ws = (enc_W1, row(enc_b1), enc_W2, row(enc_b2), enc_W3, row(enc_b3),
              row(enc_g), row(enc_bn))

    ea = attr
    for i in range(2):
        W1 = edge_W1[i]
        proj = _project(x_dst, x_src, W1[0:D, :], W1[D:2 * D, :],
                        row(edge_b1[i]))
        table = proj.reshape(2 * N, D)
        e_new, parts = [], []
        for h in range(NSP):
            s = _gather_sum(table, dix[h], six[h])
            e_new.append(_edge_mlp(
                s, ea[h], enc_ws, W1[2 * D:3 * D, :], edge_W2[i], edge_W3[i],
                row(edge_b2[i]), row(edge_b3[i]), row(edge_g[i]),
                row(edge_bn[i]), first_layer=(i == 0),
            ))
            parts.append(_scatter_add(e_new[h], dix[h], zeros_pad))
        x_dst = _node_mlp(
            x_dst, [p[:, 0:N, :] for p in parts], node_W1[i], row(node_b1[i]),
            node_W2[i], row(node_b2[i]), node_W3[i], row(node_b3[i]),
            row(node_g[i]), row(node_bn[i]),
        )
        ea = e_new
    return x_dst
